# baseline (speedup 1.0000x reference)
import functools

import jax
import jax.numpy as jnp
from jax import lax
from jax.experimental import pallas as pl
from jax.experimental.pallas import tpu as pltpu
from jax.experimental.pallas import tpu_sc as plsc

SEQ_LEN = 1024
EMBED_DIM = 16
VOCAB = 128
BLOCK = 128
DHALF = EMBED_DIM // 2
_NBLOCKS = SEQ_LEN // BLOCK

_L = plsc.get_sparse_core_info().num_lanes

_mesh = plsc.VectorSubcoreMesh(
    core_axis_name="c", subcore_axis_name="s", num_cores=1)


@functools.partial(
    pl.kernel,
    mesh=_mesh,
    out_type=jax.ShapeDtypeStruct((EMBED_DIM, SEQ_LEN), jnp.float32),
    compiler_params=pltpu.CompilerParams(needs_layout_passes=False),
    scratch_types=[
        pltpu.VMEM((BLOCK,), jnp.int32),
        pltpu.VMEM((DHALF, VOCAB), jnp.float32),
        pltpu.VMEM((DHALF, BLOCK), jnp.float32),
        pltpu.SemaphoreType.DMA,
        pltpu.SemaphoreType.DMA,
        pltpu.SemaphoreType.DMA,
    ],
)
def _embed_add(idx_hbm, tok_hbm, pos_hbm, out_hbm, idx_v, tok_v,
               out_v, idx_sem, tok_sem, pos_sem):
    wid = lax.axis_index("s") + lax.axis_index("c")
    blk = wid & (_NBLOCKS - 1)
    drow = pl.multiple_of((wid >> 3) * DHALF, DHALF)
    base = blk * BLOCK
    idx_cp = pltpu.async_copy(idx_hbm.at[pl.ds(base, BLOCK)], idx_v, idx_sem)
    tok_cp = pltpu.async_copy(tok_hbm.at[pl.ds(drow, DHALF)], tok_v, tok_sem)
    pos_cp = pltpu.async_copy(
        pos_hbm.at[pl.ds(drow, DHALF), pl.ds(base, BLOCK)], out_v, pos_sem)
    idx_cp.wait()
    tok_cp.wait()
    pos_cp.wait()

    def body(g, carry):
        off = g * _L
        tok_idx = idx_v[pl.ds(off, _L)]
        for d in range(DHALF):
            dvec = jnp.full((_L,), d, jnp.int32)
            vals = plsc.load_gather(tok_v, [dvec, tok_idx])
            plsc.addupdate(out_v.at[d, pl.ds(off, _L)], vals)
        return carry

    lax.fori_loop(0, BLOCK // _L, body, 0)
    pltpu.sync_copy(
        out_v, out_hbm.at[pl.ds(drow, DHALF), pl.ds(base, BLOCK)])


def kernel(encoded, token_table, pos_table):
    out_t = _embed_add(encoded.astype(jnp.int32), token_table.T, pos_table.T)
    return out_t.T


# speedup vs baseline: 1.0020x; 1.0020x over previous
import functools

import jax
import jax.numpy as jnp
from jax import lax
from jax.experimental import pallas as pl
from jax.experimental.pallas import tpu as pltpu
from jax.experimental.pallas import tpu_sc as plsc

SEQ_LEN = 1024
EMBED_DIM = 16
VOCAB = 128
BLOCK = 128
DHALF = EMBED_DIM // 2
_NBLOCKS = SEQ_LEN // BLOCK

_L = plsc.get_sparse_core_info().num_lanes

_mesh = plsc.VectorSubcoreMesh(
    core_axis_name="c", subcore_axis_name="s", num_cores=1)


@functools.partial(
    pl.kernel,
    mesh=_mesh,
    out_type=jax.ShapeDtypeStruct((EMBED_DIM, SEQ_LEN), jnp.float32),
    compiler_params=pltpu.CompilerParams(
        needs_layout_passes=False,
        disable_bounds_checks=True,
        disable_semaphore_checks=True,
    ),
    scratch_types=[
        pltpu.VMEM((BLOCK,), jnp.int32),
        pltpu.VMEM((DHALF, VOCAB), jnp.float32),
        pltpu.VMEM((DHALF, BLOCK), jnp.float32),
        pltpu.SemaphoreType.DMA,
        pltpu.SemaphoreType.DMA,
        pltpu.SemaphoreType.DMA,
    ],
)
def _embed_add(idx_hbm, tok_hbm, pos_hbm, out_hbm, idx_v, tok_v,
               out_v, idx_sem, tok_sem, pos_sem):
    wid = lax.axis_index("s") + lax.axis_index("c")
    blk = wid & (_NBLOCKS - 1)
    drow = pl.multiple_of((wid >> 3) * DHALF, DHALF)
    base = blk * BLOCK
    idx_cp = pltpu.async_copy(idx_hbm.at[pl.ds(base, BLOCK)], idx_v, idx_sem)
    tok_cp = pltpu.async_copy(tok_hbm.at[pl.ds(drow, DHALF)], tok_v, tok_sem)
    pos_cp = pltpu.async_copy(
        pos_hbm.at[pl.ds(drow, DHALF), pl.ds(base, BLOCK)], out_v, pos_sem)
    idx_cp.wait()
    tok_cp.wait()
    pos_cp.wait()

    def body(g, carry):
        off = g * _L
        tok_idx = idx_v[pl.ds(off, _L)]
        for d in range(DHALF):
            dvec = jnp.full((_L,), d, jnp.int32)
            vals = plsc.load_gather(tok_v, [dvec, tok_idx])
            plsc.addupdate(out_v.at[d, pl.ds(off, _L)], vals)
        return carry

    lax.fori_loop(0, BLOCK // _L, body, 0)
    pltpu.sync_copy(
        out_v, out_hbm.at[pl.ds(drow, DHALF), pl.ds(base, BLOCK)])


def kernel(encoded, token_table, pos_table):
    out_t = _embed_add(encoded.astype(jnp.int32), token_table.T, pos_table.T)
    return out_t.T


# speedup vs baseline: 1.0116x; 1.0095x over previous
import functools

import jax
import jax.numpy as jnp
from jax import lax
from jax.experimental import pallas as pl
from jax.experimental.pallas import tpu as pltpu
from jax.experimental.pallas import tpu_sc as plsc

SEQ_LEN = 1024
EMBED_DIM = 16
VOCAB = 128
BLOCK = 128
DHALF = EMBED_DIM // 2
_NBLOCKS = SEQ_LEN // BLOCK

_L = plsc.get_sparse_core_info().num_lanes

_mesh = plsc.VectorSubcoreMesh(
    core_axis_name="c", subcore_axis_name="s", num_cores=1)


@functools.partial(
    pl.kernel,
    mesh=_mesh,
    out_type=jax.ShapeDtypeStruct((EMBED_DIM, SEQ_LEN), jnp.float32),
    compiler_params=pltpu.CompilerParams(needs_layout_passes=False),
    scratch_types=[
        pltpu.VMEM((BLOCK,), jnp.int32),
        pltpu.VMEM((DHALF, VOCAB), jnp.float32),
        pltpu.VMEM((DHALF, BLOCK), jnp.float32),
        pltpu.SemaphoreType.DMA,
        pltpu.SemaphoreType.DMA,
        pltpu.SemaphoreType.DMA,
    ],
)
def _embed_add(idx_hbm, tok_hbm, pos_hbm, out_hbm, idx_v, tok_v,
               out_v, idx_sem, tok_sem, pos_sem):
    wid = lax.axis_index("s") + lax.axis_index("c")
    blk = wid & (_NBLOCKS - 1)
    drow = pl.multiple_of((wid >> 3) * DHALF, DHALF)
    base = blk * BLOCK
    idx_cp = pltpu.async_copy(idx_hbm.at[pl.ds(base, BLOCK)], idx_v, idx_sem)
    tok_cp = pltpu.async_copy(tok_hbm.at[pl.ds(drow, DHALF)], tok_v, tok_sem)
    pos_cp = pltpu.async_copy(
        pos_hbm.at[pl.ds(drow, DHALF), pl.ds(base, BLOCK)], out_v, pos_sem)
    idx_cp.wait()
    tok_cp.wait()
    pos_cp.wait()

    def body(g, carry):
        off = g * _L
        tok_idx = idx_v[pl.ds(off, _L)]
        for d in range(DHALF):
            dvec = jnp.full((_L,), d, jnp.int32)
            vals = plsc.load_gather(tok_v, [dvec, tok_idx])
            plsc.addupdate(out_v.at[d, pl.ds(off, _L)], vals)
        return carry

    lax.fori_loop(0, BLOCK // _L, body, 0)
    pltpu.sync_copy(
        out_v, out_hbm.at[pl.ds(drow, DHALF), pl.ds(base, BLOCK)])


def kernel(encoded, token_table, pos_table):
    out_t = _embed_add(encoded.astype(jnp.int32), token_table.T, pos_table.T)
    return out_t.T
